# baseline (speedup 1.0000x reference)
import jax
import jax.numpy as jnp
from jax import lax
from jax.experimental import pallas as pl
from jax.experimental.pallas import tpu as pltpu

_THR = 0.8
_N = 1024


def _filter_kernel(raw_ref, tr_ref, num_ref, out_ref, nk_ref):
    n = num_ref[0, 0, 0]
    raw = raw_ref[0]
    tr = tr_ref[0]
    x1r, y1r, x2r, y2r = tr[0:1, :], tr[1:2, :], tr[2:3, :], tr[3:4, :]
    x1c, y1c, x2c, y2c = raw[:, 0:1], raw[:, 1:2], raw[:, 2:3], raw[:, 3:4]
    area_r = (x2r - x1r) * (y2r - y1r)
    area_c = (x2c - x1c) * (y2c - y1c)
    ir = lax.broadcasted_iota(jnp.int32, (1, _N), 1)
    ic = lax.broadcasted_iota(jnp.int32, (_N, 1), 0)
    vr = ir < n
    vc = ic < n
    d = (x1c >= x1r) & (y1c >= y1r) & (x2c <= x2r) & (y2c <= y2r)
    d = d & vr & vc & (ic != ir)
    w = jnp.where(d, jnp.broadcast_to(area_c, (_N, _N)), 0.0)
    sum_contained = jnp.sum(w, axis=0, keepdims=True)
    keep = (sum_contained <= _THR * (area_r + 1e-9)) & vr
    keep_f = keep.astype(jnp.float32)
    le = (ic <= ir).astype(jnp.float32)
    cum = jnp.dot(keep_f, le, preferred_element_type=jnp.float32)
    pos = cum - 1.0
    p_mat = jnp.where((pos == ic.astype(jnp.float32)) & keep, 1.0, 0.0)
    filt = jnp.dot(p_mat, raw, preferred_element_type=jnp.float32,
                   precision=lax.Precision.HIGHEST)
    nk = jnp.sum(keep_f).astype(jnp.int32)
    out_ref[0] = jnp.where(nk > 0, filt, raw)
    nk_ref[0, 0, 0] = nk


def kernel(box_prompts, num_boxes):
    T, C, MAXB, F = box_prompts.shape
    S = T * C
    raw = box_prompts.reshape(S, MAXB, F)
    raw = jnp.pad(raw, ((0, 0), (0, _N - MAXB), (0, 0)))
    tr = raw.transpose(0, 2, 1)
    num = num_boxes.reshape(S, 1, 1)
    out, nk = pl.pallas_call(
        _filter_kernel,
        grid=(S,),
        in_specs=[
            pl.BlockSpec((1, _N, F), lambda i: (i, 0, 0)),
            pl.BlockSpec((1, F, _N), lambda i: (i, 0, 0)),
            pl.BlockSpec((1, 1, 1), lambda i: (i, 0, 0), memory_space=pltpu.SMEM),
        ],
        out_specs=[
            pl.BlockSpec((1, _N, F), lambda i: (i, 0, 0)),
            pl.BlockSpec((1, 1, 1), lambda i: (i, 0, 0), memory_space=pltpu.SMEM),
        ],
        out_shape=[
            jax.ShapeDtypeStruct((S, _N, F), jnp.float32),
            jax.ShapeDtypeStruct((S, 1, 1), jnp.int32),
        ],
        compiler_params=pltpu.CompilerParams(
            dimension_semantics=("parallel",)
        ),
    )(raw, tr, num)
    filtered = out[:, :MAXB, :].reshape(T, C, MAXB, F)
    return filtered, nk.reshape(T, C)


# speedup vs baseline: 1.3655x; 1.3655x over previous
import jax
import jax.numpy as jnp
from jax import lax
from jax.experimental import pallas as pl
from jax.experimental.pallas import tpu as pltpu

_THR = 0.8
_N = 1024


def _filter_kernel(raw_ref, tr_ref, num_ref, out_ref, nk_ref):
    n = num_ref[0, 0, 0]
    raw = raw_ref[0]
    tr = tr_ref[0]
    x1r, y1r, x2r, y2r = tr[0:1, :], tr[1:2, :], tr[2:3, :], tr[3:4, :]
    x1c, y1c, x2c, y2c = raw[:, 0:1], raw[:, 1:2], raw[:, 2:3], raw[:, 3:4]
    area_r = (x2r - x1r) * (y2r - y1r)
    area_c = (x2c - x1c) * (y2c - y1c)
    ir = lax.broadcasted_iota(jnp.int32, (1, _N), 1)
    ic = lax.broadcasted_iota(jnp.int32, (_N, 1), 0)
    vr = ir < n
    vc = ic < n
    d = (x1c >= x1r) & (y1c >= y1r) & (x2c <= x2r) & (y2c <= y2r)
    d = d & vr & vc & (ic != ir)
    w = jnp.where(d, jnp.broadcast_to(area_c, (_N, _N)), 0.0)
    sum_contained = jnp.sum(w, axis=0, keepdims=True)
    keep = (sum_contained <= _THR * (area_r + 1e-9)) & vr
    keep_f = keep.astype(jnp.float32)
    cum = keep_f
    sh = 1
    while sh < _N:
        rolled = pltpu.roll(cum, sh, 1)
        cum = cum + jnp.where(ir >= sh, rolled, 0.0)
        sh *= 2
    pos = cum - 1.0
    p_mat = jnp.where((pos == ic.astype(jnp.float32)) & keep, 1.0, 0.0)
    hi = raw.astype(jnp.bfloat16).astype(jnp.float32)
    lo = raw - hi
    filt = (jnp.dot(p_mat, hi, preferred_element_type=jnp.float32)
            + jnp.dot(p_mat, lo, preferred_element_type=jnp.float32))
    nk = jnp.sum(keep_f).astype(jnp.int32)
    out_ref[0] = jnp.where(nk > 0, filt, raw)
    nk_ref[0, 0, 0] = nk


def kernel(box_prompts, num_boxes):
    T, C, MAXB, F = box_prompts.shape
    S = T * C
    raw = box_prompts.reshape(S, MAXB, F)
    raw = jnp.pad(raw, ((0, 0), (0, _N - MAXB), (0, 0)))
    tr = raw.transpose(0, 2, 1)
    num = num_boxes.reshape(S, 1, 1)
    out, nk = pl.pallas_call(
        _filter_kernel,
        grid=(S,),
        in_specs=[
            pl.BlockSpec((1, _N, F), lambda i: (i, 0, 0)),
            pl.BlockSpec((1, F, _N), lambda i: (i, 0, 0)),
            pl.BlockSpec((1, 1, 1), lambda i: (i, 0, 0), memory_space=pltpu.SMEM),
        ],
        out_specs=[
            pl.BlockSpec((1, _N, F), lambda i: (i, 0, 0)),
            pl.BlockSpec((1, 1, 1), lambda i: (i, 0, 0), memory_space=pltpu.SMEM),
        ],
        out_shape=[
            jax.ShapeDtypeStruct((S, _N, F), jnp.float32),
            jax.ShapeDtypeStruct((S, 1, 1), jnp.int32),
        ],
        compiler_params=pltpu.CompilerParams(
            dimension_semantics=("parallel",)
        ),
    )(raw, tr, num)
    filtered = out[:, :MAXB, :].reshape(T, C, MAXB, F)
    return filtered, nk.reshape(T, C)


# speedup vs baseline: 2.5604x; 1.8751x over previous
import jax
import jax.numpy as jnp
from jax import lax
from jax.experimental import pallas as pl
from jax.experimental.pallas import tpu as pltpu

_THR = 0.8
_N = 1024
_CH = 128


def _filter_kernel(raw_ref, tr_ref, num_ref, out_ref, nk_ref):
    n = num_ref[0, 0, 0]
    tr = tr_ref[0]
    x1r, y1r, x2r, y2r = tr[0:1, :], tr[1:2, :], tr[2:3, :], tr[3:4, :]
    area_r = (x2r - x1r) * (y2r - y1r)
    ir = lax.broadcasted_iota(jnp.int32, (1, _N), 1)
    vr = ir < n
    icc = lax.broadcasted_iota(jnp.int32, (_CH, 1), 0)

    def body(ci, acc):
        a0 = ci * _CH
        ch = raw_ref[0, pl.ds(a0, _CH), :]
        x1c, y1c = ch[:, 0:1], ch[:, 1:2]
        x2c, y2c = ch[:, 2:3], ch[:, 3:4]
        area_c = (x2c - x1c) * (y2c - y1c)
        vc = (icc + a0) < n
        d = (x1c >= x1r) & (y1c >= y1r) & (x2c <= x2r) & (y2c <= y2r)
        d = d & vc
        w = jnp.where(d, jnp.broadcast_to(area_c, (_CH, _N)), 0.0)
        return acc + jnp.sum(w, axis=0, keepdims=True)

    nch = (n + _CH - 1) // _CH
    sum_self = lax.fori_loop(0, nch, body, jnp.zeros((1, _N), jnp.float32))
    keep = (sum_self <= (1.0 + _THR) * area_r + _THR * 1e-9) & vr
    keep_f = keep.astype(jnp.float32)

    cum = keep_f
    sh = 1
    while sh < _N:
        cum = cum + jnp.where(ir >= sh, pltpu.roll(cum, sh, 1), 0.0)
        sh *= 2
    nk = jnp.sum(keep_f).astype(jnp.int32)

    vals = tr
    occ = keep.astype(jnp.int32)
    r = ir - (cum.astype(jnp.int32) - 1)
    sh = 1
    b = 0
    while sh < _N:
        occ_s = pltpu.roll(occ, _N - sh, 1)
        r_s = pltpu.roll(r, _N - sh, 1)
        vals_s = pltpu.roll(vals, _N - sh, 1)
        bit_s = lax.shift_right_logical(r_s, b) & 1
        arriving = ((occ_s & bit_s) == 1) & (ir < _N - sh)
        bit = lax.shift_right_logical(r, b) & 1
        staying = (occ & (1 - bit)) == 1
        vals = jnp.where(arriving, vals_s, vals)
        r = jnp.where(arriving, r_s - sh, r)
        occ = jnp.where(arriving | staying, 1, 0)
        sh *= 2
        b += 1

    compacted = jnp.where(ir < nk, vals, 0.0)
    out_ref[0] = jnp.where(nk > 0, compacted, tr)
    nk_ref[0, 0, 0] = nk


def kernel(box_prompts, num_boxes):
    T, C, MAXB, F = box_prompts.shape
    S = T * C
    raw = box_prompts.reshape(S, MAXB, F)
    raw = jnp.pad(raw, ((0, 0), (0, _N - MAXB), (0, 0)))
    tr = raw.transpose(0, 2, 1)
    num = num_boxes.reshape(S, 1, 1)
    out, nk = pl.pallas_call(
        _filter_kernel,
        grid=(S,),
        in_specs=[
            pl.BlockSpec((1, _N, F), lambda i: (i, 0, 0)),
            pl.BlockSpec((1, F, _N), lambda i: (i, 0, 0)),
            pl.BlockSpec((1, 1, 1), lambda i: (i, 0, 0), memory_space=pltpu.SMEM),
        ],
        out_specs=[
            pl.BlockSpec((1, F, _N), lambda i: (i, 0, 0)),
            pl.BlockSpec((1, 1, 1), lambda i: (i, 0, 0), memory_space=pltpu.SMEM),
        ],
        out_shape=[
            jax.ShapeDtypeStruct((S, F, _N), jnp.float32),
            jax.ShapeDtypeStruct((S, 1, 1), jnp.int32),
        ],
        compiler_params=pltpu.CompilerParams(
            dimension_semantics=("parallel",)
        ),
    )(raw, tr, num)
    filtered = out.transpose(0, 2, 1)[:, :MAXB, :].reshape(T, C, MAXB, F)
    return filtered, nk.reshape(T, C)
